# baseline (speedup 1.0000x reference)
import jax
import jax.numpy as jnp
import numpy as np
from jax import lax
from jax.experimental import pallas as pl
from jax.experimental.pallas import tpu as pltpu
from jax.experimental.pallas import tpu_sc as plsc

VOCAB = 100000
D_MODEL = 768
SEQ_LEN = 4096
BATCH = 4

NUM_CORES = 2
NUM_SUBCORES = 16
NUM_WORKERS = NUM_CORES * NUM_SUBCORES

POS_PER_W = SEQ_LEN // NUM_WORKERS
CHUNK = 16
N_CHUNKS = POS_PER_W // CHUNK
LANES = 16
D_VECS = D_MODEL // LANES


def _pos_encoding_np(max_len: int, d_model: int) -> np.ndarray:
    pos = np.arange(max_len, dtype=np.float32)[:, None]
    _2i = np.arange(0, d_model, 2, dtype=np.float32)
    enc = np.zeros((max_len, d_model), dtype=np.float32)
    angle = pos / np.power(np.float32(10000.0), _2i / np.float32(d_model))
    enc[:, 0::2] = np.sin(angle)
    enc[:, 1::2] = np.cos(angle)
    return enc


_POS_ENC = _pos_encoding_np(SEQ_LEN, D_MODEL)


def _sc_body(x_hbm, pos_hbm, table_hbm, out_hbm,
             idx_v, bufs, pos_v, gsem0, gsem1, osem0, osem1):
    wid = lax.axis_index("s") * NUM_CORES + lax.axis_index("c")
    p0 = wid * POS_PER_W

    for b in range(BATCH):
        pltpu.sync_copy(x_hbm.at[b, wid], idx_v.at[b])

    gsems = (gsem0, gsem1)
    osems = (osem0, osem1)

    def fire_gathers(j, s):
        return [
            pltpu.async_copy(table_hbm.at[idx_v.at[b, j]], bufs.at[s, b],
                             gsems[s])
            for b in range(BATCH)
        ]

    def fire_writebacks(j, s):
        return [
            pltpu.async_copy(
                bufs.at[s, b],
                out_hbm.at[pl.ds(b * SEQ_LEN + p0 + j * CHUNK, CHUNK)],
                osems[s])
            for b in range(BATCH)
        ]

    def add_chunk(s):
        def rbody(r, _):
            for half in range(2):
                base = half * (D_VECS // 2) * LANES
                prow = [pos_v[r, pl.ds(base + c * LANES, LANES)]
                        for c in range(D_VECS // 2)]
                for b in range(BATCH):
                    for c in range(D_VECS // 2):
                        sl = pl.ds(base + c * LANES, LANES)
                        bufs[s, b, r, sl] = bufs[s, b, r, sl] + prow[c]
            return 0
        lax.fori_loop(0, CHUNK, rbody, 0)

    ga = [None, None]
    ob = [None, None]
    ga[0] = fire_gathers(0, 0)
    for j in range(N_CHUNKS):
        s = j % 2
        t = (j + 1) % 2
        if j + 1 < N_CHUNKS:
            if ob[t] is not None:
                for o in ob[t]:
                    o.wait()
            ga[t] = fire_gathers(j + 1, t)
        for g in ga[s]:
            g.wait()
        pltpu.sync_copy(pos_hbm.at[pl.ds(p0 + j * CHUNK, CHUNK)], pos_v)
        add_chunk(s)
        ob[s] = fire_writebacks(j, s)
    for s in range(2):
        for o in ob[s]:
            o.wait()


@jax.jit
def _embed(x_r, emb_table, pos_enc):
    mesh = plsc.VectorSubcoreMesh(core_axis_name="c", subcore_axis_name="s")
    run = pl.kernel(
        _sc_body,
        out_type=jax.ShapeDtypeStruct((BATCH * SEQ_LEN, D_MODEL), jnp.float32),
        mesh=mesh,
        scratch_types=[
            pltpu.VMEM((BATCH, N_CHUNKS, CHUNK), jnp.int32),
            pltpu.VMEM((2, BATCH, CHUNK, D_MODEL), jnp.float32),
            pltpu.VMEM((CHUNK, D_MODEL), jnp.float32),
            pltpu.SemaphoreType.DMA,
            pltpu.SemaphoreType.DMA,
            pltpu.SemaphoreType.DMA,
            pltpu.SemaphoreType.DMA,
        ],
    )
    return run(x_r, pos_enc, emb_table)


def kernel(x, emb_table):
    x_r = x.reshape(BATCH, NUM_WORKERS, N_CHUNKS, CHUNK).astype(jnp.int32)
    pos_enc = jnp.asarray(_POS_ENC)
    out = _embed(x_r, emb_table, pos_enc)
    return out.reshape(BATCH, SEQ_LEN, D_MODEL)
